# baseline (speedup 1.0000x reference)
import functools

import jax
import jax.numpy as jnp
import numpy as np
from jax.experimental import pallas as pl
from jax.experimental.pallas import tpu as pltpu
from jax.experimental.shard_map import shard_map
from jax.sharding import Mesh, PartitionSpec as P

_NL = 128


def _balanced_add(ts):
    n = len(ts)
    if n == 1:
        return ts[0]
    return _balanced_add(ts[: n // 2]) + _balanced_add(ts[n // 2:])


def _dw_accs(x_ref, dw_ref, xp_ref):
    cin, h, w, _ = x_ref.shape
    xp_ref[:, :, 0:1, :] = jnp.zeros_like(xp_ref[:, :, 0:1, :])
    xp_ref[:, :, w + 1:w + 2, :] = jnp.zeros_like(xp_ref[:, :, w + 1:w + 2, :])
    xp_ref[:, 0:1, :, :] = jnp.zeros_like(xp_ref[:, 0:1, :, :])
    xp_ref[:, h + 1:h + 2, :, :] = jnp.zeros_like(xp_ref[:, h + 1:h + 2, :, :])
    xp_ref[:, 1:h + 1, 1:w + 1, :] = jnp.maximum(x_ref[...], 0.0)

    accs = []
    for ci in range(cin):
        taps = [xp_ref[ci, kh:kh + h, kw:kw + w, :] * dw_ref[ci, kh * 3 + kw]
                for kh in range(3) for kw in range(3)]
        accs.append(_balanced_add(taps))
    return accs


def _conv_ys(x_ref, dw_ref, pm_ref, xp_ref):
    cin = x_ref.shape[0]
    cout = pm_ref.shape[0]
    accs = _dw_accs(x_ref, dw_ref, xp_ref)
    ys = [None] * cout
    for ci in range(cin):
        for co in range(cout):
            t = accs[ci] * pm_ref[co, ci]
            ys[co] = t if ci == 0 else ys[co] + t
    return ys


def _moments_kernel(x_ref, dw_ref, pm_ref, mom_ref, xp_ref):
    ys = _conv_ys(x_ref, dw_ref, pm_ref, xp_ref)
    for co, y in enumerate(ys):
        mom_ref[0, co, 0] = jnp.sum(y, axis=(0, 1))
        mom_ref[0, co, 1] = jnp.sum(y * y, axis=(0, 1))


def _bn_apply_kernel(x_ref, dw_ref, pm_ref, shift_ref, o_ref, xp_ref):
    ys = _conv_ys(x_ref, dw_ref, pm_ref, xp_ref)
    for co, y in enumerate(ys):
        o_ref[co] = y + shift_ref[co, 0]


def _sep_conv_sharded(xt, dw, pmat, gamma, beta, *, n_total, axis=None):
    cin, h, w, n_loc = xt.shape
    cout = pmat.shape[0]
    nl = _NL
    grid = (n_loc // nl,)
    eps = 1e-5

    cparams = pltpu.CompilerParams(dimension_semantics=("parallel",),
                                   vmem_limit_bytes=64 * 1024 * 1024)
    smem = pl.BlockSpec(memory_space=pltpu.SMEM)

    moments = pl.pallas_call(
        _moments_kernel,
        out_shape=jax.ShapeDtypeStruct((n_loc // nl, cout, 2, nl), jnp.float32),
        grid=grid,
        in_specs=[pl.BlockSpec((cin, h, w, nl), lambda i: (0, 0, 0, i)),
                  smem, smem],
        out_specs=pl.BlockSpec((1, cout, 2, nl), lambda i: (i, 0, 0, 0)),
        scratch_shapes=[pltpu.VMEM((cin, h + 2, w + 2, nl), jnp.float32)],
        compiler_params=cparams,
    )(xt, dw, pmat)

    tot = jnp.sum(moments, axis=(0, 3))
    if axis is not None:
        tot = jax.lax.psum(tot, axis)
    count = jnp.float32(n_total * h * w)
    mean = tot[:, 0:1] / count
    var = tot[:, 1:2] / count - mean * mean
    inv = jax.lax.rsqrt(var + eps)
    scale = gamma.reshape(cout, 1) * inv
    shift = beta.reshape(cout, 1) - mean * scale
    pmat_s = pmat * scale

    return pl.pallas_call(
        _bn_apply_kernel,
        out_shape=jax.ShapeDtypeStruct((cout, h, w, n_loc), jnp.float32),
        grid=grid,
        in_specs=[pl.BlockSpec((cin, h, w, nl), lambda i: (0, 0, 0, i)),
                  smem, smem, smem],
        out_specs=pl.BlockSpec((cout, h, w, nl), lambda i: (0, 0, 0, i)),
        scratch_shapes=[pltpu.VMEM((cin, h + 2, w + 2, nl), jnp.float32)],
        compiler_params=cparams,
    )(xt, dw, pmat_s, shift)


def kernel(x_nchw, dw_w, pw_w, gamma, beta):
    n, cin, h, w = x_nchw.shape
    cout = pw_w.shape[0]

    xt = jnp.transpose(x_nchw.astype(jnp.float32), (1, 2, 3, 0))
    dw = dw_w.astype(jnp.float32).reshape(cin, 9)
    pmat = pw_w.astype(jnp.float32).reshape(cout, cin)
    gamma = gamma.astype(jnp.float32)
    beta = beta.astype(jnp.float32)

    devs = jax.devices()
    nd = 2 if len(devs) >= 2 and n % (2 * _NL) == 0 else 1
    if nd == 1:
        yt = _sep_conv_sharded(xt, dw, pmat, gamma, beta, n_total=n)
    else:
        mesh = Mesh(np.array(devs[:nd]), ("d",))
        run = shard_map(
            functools.partial(_sep_conv_sharded, n_total=n, axis="d"),
            mesh=mesh,
            in_specs=(P(None, None, None, "d"), P(None, None), P(None, None),
                      P(None), P(None)),
            out_specs=P(None, None, None, "d"),
            check_rep=False,
        )
        yt = run(xt, dw, pmat, gamma, beta)

    return jnp.transpose(yt, (3, 0, 1, 2))


# speedup vs baseline: 4.6218x; 4.6218x over previous
import jax
import jax.numpy as jnp
from jax.experimental import pallas as pl
from jax.experimental.pallas import tpu as pltpu

_NL = 128


def _balanced_add(ts):
    n = len(ts)
    if n == 1:
        return ts[0]
    return _balanced_add(ts[: n // 2]) + _balanced_add(ts[n // 2:])


def _pairs(cin):
    return [(i, j) for i in range(cin) for j in range(i, cin)]


def _dw_accs(x_ref, dw_ref, xp_ref):
    cin, h, w, _ = x_ref.shape
    xp_ref[:, :, 0:1, :] = jnp.zeros_like(xp_ref[:, :, 0:1, :])
    xp_ref[:, :, w + 1:w + 2, :] = jnp.zeros_like(xp_ref[:, :, w + 1:w + 2, :])
    xp_ref[:, 0:1, :, :] = jnp.zeros_like(xp_ref[:, 0:1, :, :])
    xp_ref[:, h + 1:h + 2, :, :] = jnp.zeros_like(xp_ref[:, h + 1:h + 2, :, :])
    xp_ref[:, 1:h + 1, 1:w + 1, :] = jnp.maximum(x_ref[...], 0.0)

    accs = []
    for ci in range(cin):
        taps = [xp_ref[ci, kh:kh + h, kw:kw + w, :] * dw_ref[ci, kh * 3 + kw]
                for kh in range(3) for kw in range(3)]
        accs.append(_balanced_add(taps))
    return accs


def _conv_ys(x_ref, dw_ref, pm_ref, xp_ref):
    cin = x_ref.shape[0]
    cout = pm_ref.shape[0]
    accs = _dw_accs(x_ref, dw_ref, xp_ref)
    ys = [None] * cout
    for ci in range(cin):
        for co in range(cout):
            t = accs[ci] * pm_ref[co, ci]
            ys[co] = t if ci == 0 else ys[co] + t
    return ys


def _moments_kernel(x_ref, dw_ref, pm_ref, mom_ref, xp_ref):
    ys = _conv_ys(x_ref, dw_ref, pm_ref, xp_ref)
    for co, y in enumerate(ys):
        mom_ref[0, co, 0] = jnp.sum(y, axis=(0, 1))
        mom_ref[0, co, 1] = jnp.sum(y * y, axis=(0, 1))


def _bn_apply_kernel(x_ref, dw_ref, pm_ref, shift_ref, o_ref, xp_ref):
    ys = _conv_ys(x_ref, dw_ref, pm_ref, xp_ref)
    for co, y in enumerate(ys):
        o_ref[co] = y + shift_ref[co, 0]


def kernel(x_nchw, dw_w, pw_w, gamma, beta):
    n, cin, h, w = x_nchw.shape
    cout = pw_w.shape[0]
    nl = _NL
    assert n % nl == 0
    grid = (n // nl,)
    eps = 1e-5

    xt = jnp.transpose(x_nchw.astype(jnp.float32), (1, 2, 3, 0))

    dw = dw_w.astype(jnp.float32).reshape(cin, 9)
    pmat = pw_w.astype(jnp.float32).reshape(cout, cin)

    cparams = pltpu.CompilerParams(dimension_semantics=("parallel",),
                                   vmem_limit_bytes=64 * 1024 * 1024)
    smem = pl.BlockSpec(memory_space=pltpu.SMEM)

    moments = pl.pallas_call(
        _moments_kernel,
        out_shape=jax.ShapeDtypeStruct((n // nl, cout, 2, nl), jnp.float32),
        grid=grid,
        in_specs=[pl.BlockSpec((cin, h, w, nl), lambda i: (0, 0, 0, i)),
                  smem, smem],
        out_specs=pl.BlockSpec((1, cout, 2, nl), lambda i: (i, 0, 0, 0)),
        scratch_shapes=[pltpu.VMEM((cin, h + 2, w + 2, nl), jnp.float32)],
        compiler_params=cparams,
    )(xt, dw, pmat)

    tot = jnp.sum(moments, axis=(0, 3))
    count = jnp.float32(n * h * w)
    mean = tot[:, 0:1] / count
    var = tot[:, 1:2] / count - mean * mean
    inv = jax.lax.rsqrt(var + eps)
    scale = gamma.astype(jnp.float32).reshape(cout, 1) * inv
    shift = beta.astype(jnp.float32).reshape(cout, 1) - mean * scale
    pmat_s = pmat * scale

    yt = pl.pallas_call(
        _bn_apply_kernel,
        out_shape=jax.ShapeDtypeStruct((cout, h, w, n), jnp.float32),
        grid=grid,
        in_specs=[pl.BlockSpec((cin, h, w, nl), lambda i: (0, 0, 0, i)),
                  smem, smem, smem],
        out_specs=pl.BlockSpec((cout, h, w, nl), lambda i: (0, 0, 0, i)),
        scratch_shapes=[pltpu.VMEM((cin, h + 2, w + 2, nl), jnp.float32)],
        compiler_params=cparams,
    )(xt, dw, pmat_s, shift)

    return jnp.transpose(yt, (3, 0, 1, 2))
